# baseline (speedup 1.0000x reference)
import jax
import jax.numpy as jnp
from jax import lax
from jax.experimental import pallas as pl
from jax.experimental.pallas import tpu as pltpu
from jax.experimental.pallas import tpu_sc as plsc

N = 50000
E = 800000
H = 128
S = 5000
G = 64

ROWS = 1000
NSTEP = N // ROWS

EPAD = 819200
NPAD = 50688
RNG = 8448
ACCR = RNG + 16
W2 = 3200
NW2 = EPAD // 16 // W2
NWTOT = EPAD // W2
CH = 64
CBUF = W2 + 272
ZR = 16
NB1 = NPAD // 32
SPAD = 5008


def _z():
    return jnp.int32(0)


def _scalar(v):
    return v if getattr(v, "ndim", 0) == 0 else jnp.max(v)


def _agg_body(h_hbm, ei_hbm, out_hbm,
              svbuf, csrc, cdst,
              cs0, cd0, cs1, cd1, cs2, cd2, cs3, cd3,
              rows0, rows1, rows2, rows3, zbuf, acc,
              gs0, gs1, gs2, gs3, ss0, ss1, ss2, ss3):
    c = lax.axis_index("c")
    s = lax.axis_index("s")
    csq = [cs0, cs1, cs2, cs3]
    cdq = [cd0, cd1, cd2, cd3]
    rowsq = [rows0, rows1, rows2, rows3]
    gsem = [gs0, gs1, gs2, gs3]
    ssem = [ss0, ss1, ss2, ss3]

    def zrow(r, _):
        for q in range(H // 16):
            zbuf[r, pl.ds(q * 16, 16)] = jnp.zeros((16,), jnp.float32)
        return jnp.int32(0)

    lax.fori_loop(jnp.int32(0), jnp.int32(ZR), zrow, jnp.int32(0))

    for p in range(3):
        base = (3 * c + p) * RNG
        for q in range(528 // ZR):
            pltpu.sync_copy(zbuf, acc.at[pl.ds(s * 528 + q * ZR, ZR)])
        plsc.subcore_barrier()

        g0 = s * NW2

        def window(w, _):
            pltpu.sync_copy(ei_hbm.at[g0 + w], svbuf)

            def compact(j, cnt):
                o = j * 32
                sv0 = svbuf[0, pl.ds(o, 16)]
                dv0 = svbuf[1, pl.ds(o, 16)]
                sv1 = svbuf[0, pl.ds(o + 16, 16)]
                dv1 = svbuf[1, pl.ds(o + 16, 16)]
                m0 = (dv0 >= base) & (dv0 < base + RNG)
                m1 = (dv1 >= base) & (dv1 < base + RNG)
                p0 = _scalar(plsc.all_reduce_population_count(m0))
                p1 = _scalar(plsc.all_reduce_population_count(m1))
                plsc.store_compressed(csrc.at[pl.ds(cnt, 16)], sv0, mask=m0)
                plsc.store_compressed(cdst.at[pl.ds(cnt, 16)], dv0 - base,
                                      mask=m0)
                c1 = cnt + p0
                plsc.store_compressed(csrc.at[pl.ds(c1, 16)], sv1, mask=m1)
                plsc.store_compressed(cdst.at[pl.ds(c1, 16)], dv1 - base,
                                      mask=m1)
                return c1 + p1

            cnt = lax.fori_loop(jnp.int32(0), jnp.int32(W2 // 32), compact,
                                jnp.int32(0))

            padsrc = s * 16 + lax.iota(jnp.int32, 16)
            paddst = jnp.zeros((16,), jnp.int32) + (RNG + s)
            for k in range(16):
                csrc[pl.ds(cnt + k * 16, 16)] = padsrc
                cdst[pl.ds(cnt + k * 16, 16)] = paddst

            nquad = (cnt + 4 * CH - 1) // (4 * CH)

            def drain(jp, _):
                off = pl.multiple_of(jp * (4 * CH), 4 * CH)
                for b in range(4):
                    for q in range(CH // 16):
                        o = off + b * CH + q * 16
                        csq[b][pl.ds(q * 16, 16)] = csrc[pl.ds(o, 16)]
                        cdq[b][pl.ds(q * 16, 16)] = cdst[pl.ds(o, 16)]
                gs = [pltpu.async_copy(h_hbm.at[csq[b]], rowsq[b], gsem[b])
                      for b in range(4)]
                ss = []
                for b in range(4):
                    gs[b].wait()
                    ss.append(pltpu.async_copy(rowsq[b], acc.at[cdq[b]],
                                               ssem[b], add=True))
                for b in range(4):
                    ss[b].wait()
                return jnp.int32(0)

            lax.fori_loop(jnp.int32(0), nquad, drain, jnp.int32(0))
            return jnp.int32(0)

        lax.fori_loop(jnp.int32(0), jnp.int32(NW2), window, jnp.int32(0))
        plsc.subcore_barrier()
        for q in range(2):
            o = s * 528 + q * 256
            for b in range(4):
                pltpu.sync_copy(acc.at[pl.ds(o + b * 64, 64)],
                                rowsq[b].at[pl.ds(0, 64)])
            for b in range(4):
                pltpu.sync_copy(rowsq[b].at[pl.ds(0, 64)],
                                out_hbm.at[pl.ds(base + o + b * 64, 64)])
        ot = s * 528 + 512
        pltpu.sync_copy(acc.at[pl.ds(ot, 16)], rowsq[0].at[pl.ds(0, 16)])
        pltpu.sync_copy(rowsq[0].at[pl.ds(0, 16)],
                        out_hbm.at[pl.ds(base + ot, 16)])
        plsc.subcore_barrier()


def _sc_agg(h, ei_win):
    mesh = plsc.VectorSubcoreMesh(core_axis_name="c", subcore_axis_name="s")
    f = pl.kernel(
        _agg_body,
        out_type=jax.ShapeDtypeStruct((NPAD, H), jnp.float32),
        mesh=mesh,
        compiler_params=pltpu.CompilerParams(needs_layout_passes=False),
        scratch_types=[
            pltpu.VMEM((2, W2), jnp.int32),
            pltpu.VMEM((CBUF,), jnp.int32),
            pltpu.VMEM((CBUF,), jnp.int32),
            pltpu.VMEM((CH,), jnp.int32),
            pltpu.VMEM((CH,), jnp.int32),
            pltpu.VMEM((CH,), jnp.int32),
            pltpu.VMEM((CH,), jnp.int32),
            pltpu.VMEM((CH,), jnp.int32),
            pltpu.VMEM((CH,), jnp.int32),
            pltpu.VMEM((CH,), jnp.int32),
            pltpu.VMEM((CH,), jnp.int32),
            pltpu.VMEM((CH, H), jnp.float32),
            pltpu.VMEM((CH, H), jnp.float32),
            pltpu.VMEM((CH, H), jnp.float32),
            pltpu.VMEM((CH, H), jnp.float32),
            pltpu.VMEM((ZR, H), jnp.float32),
            pltpu.VMEM_SHARED((ACCR, H), jnp.float32),
            pltpu.SemaphoreType.DMA,
            pltpu.SemaphoreType.DMA,
            pltpu.SemaphoreType.DMA,
            pltpu.SemaphoreType.DMA,
            pltpu.SemaphoreType.DMA,
            pltpu.SemaphoreType.DMA,
            pltpu.SemaphoreType.DMA,
            pltpu.SemaphoreType.DMA,
        ],
    )
    return f(h, ei_win)


def _n2g_body(n2s_hbm, s2g_hbm, n2g_hbm, idv, ogv, sgv):
    c = lax.axis_index("c")
    s = lax.axis_index("s")
    wid = s * 2 + c
    pltpu.sync_copy(s2g_hbm, sgv)
    pltpu.sync_copy(n2s_hbm.at[pl.ds(wid * NB1, NB1)], idv)

    def g(j, _):
        ids = idv[pl.ds(j * 16, 16)]
        ogv[pl.ds(j * 16, 16)] = plsc.load_gather(sgv, [ids])
        return jnp.int32(0)

    lax.fori_loop(jnp.int32(0), jnp.int32(NB1 // 16), g, jnp.int32(0))
    pltpu.sync_copy(ogv, n2g_hbm.at[pl.ds(wid * NB1, NB1)])


def _sc_n2g(n2s_pad, s2g_pad):
    mesh = plsc.VectorSubcoreMesh(core_axis_name="c", subcore_axis_name="s")
    f = pl.kernel(
        _n2g_body,
        out_type=jax.ShapeDtypeStruct((NPAD,), jnp.int32),
        mesh=mesh,
        compiler_params=pltpu.CompilerParams(needs_layout_passes=False),
        scratch_types=[
            pltpu.VMEM((NB1,), jnp.int32),
            pltpu.VMEM((NB1,), jnp.int32),
            pltpu.VMEM((SPAD,), jnp.int32),
        ],
    )
    return f(n2s_pad, s2g_pad)


def _xw_body(x_ref, w1_ref, o_ref):
    o_ref[...] = jnp.dot(x_ref[...], w1_ref[...],
                         preferred_element_type=jnp.float32)


def _xw(x, W1p):
    return pl.pallas_call(
        _xw_body,
        grid=(NSTEP,),
        in_specs=[
            pl.BlockSpec((ROWS, 2), lambda i: (i, _z())),
            pl.BlockSpec((2, H), lambda i: (_z(), _z())),
        ],
        out_specs=pl.BlockSpec((ROWS, H), lambda i: (i, _z())),
        out_shape=jax.ShapeDtypeStruct((N, H), jnp.float32),
    )(x, W1p)


def _mlp_pre_body(y_ref, a_ref, b1_ref, w2_ref, b2_ref, o_ref):
    h = jnp.maximum(y_ref[...] + a_ref[...] + b1_ref[...], 0.0)
    h = jnp.dot(h, w2_ref[...], preferred_element_type=jnp.float32) + b2_ref[...]
    o_ref[...] = jnp.maximum(h, 0.0)


def _mlp_pre(y0, agg, b1, W2p, b2):
    return pl.pallas_call(
        _mlp_pre_body,
        grid=(NSTEP,),
        in_specs=[
            pl.BlockSpec((ROWS, H), lambda i: (i, _z())),
            pl.BlockSpec((ROWS, H), lambda i: (i, _z())),
            pl.BlockSpec((1, H), lambda i: (_z(), _z())),
            pl.BlockSpec((H, H), lambda i: (_z(), _z())),
            pl.BlockSpec((1, H), lambda i: (_z(), _z())),
        ],
        out_specs=pl.BlockSpec((ROWS, H), lambda i: (i, _z())),
        out_shape=jax.ShapeDtypeStruct((N, H), jnp.float32),
    )(y0, agg, b1.reshape(1, H), W2p, b2.reshape(1, H))


def _mlp_body(h_ref, a_ref, w1_ref, b1_ref, w2_ref, b2_ref, o_ref):
    h = h_ref[...] + a_ref[...]
    h = jnp.dot(h, w1_ref[...], preferred_element_type=jnp.float32) + b1_ref[...]
    h = jnp.maximum(h, 0.0)
    h = jnp.dot(h, w2_ref[...], preferred_element_type=jnp.float32) + b2_ref[...]
    o_ref[...] = jnp.maximum(h, 0.0)


def _mlp(h, agg, W1p, b1, W2p, b2):
    return pl.pallas_call(
        _mlp_body,
        grid=(NSTEP,),
        in_specs=[
            pl.BlockSpec((ROWS, H), lambda i: (i, _z())),
            pl.BlockSpec((ROWS, H), lambda i: (i, _z())),
            pl.BlockSpec((H, H), lambda i: (_z(), _z())),
            pl.BlockSpec((1, H), lambda i: (_z(), _z())),
            pl.BlockSpec((H, H), lambda i: (_z(), _z())),
            pl.BlockSpec((1, H), lambda i: (_z(), _z())),
        ],
        out_specs=pl.BlockSpec((ROWS, H), lambda i: (i, _z())),
        out_shape=jax.ShapeDtypeStruct((N, H), jnp.float32),
    )(h, agg, W1p, b1.reshape(1, H), W2p, b2.reshape(1, H))


def _mlp_pool_body(h_ref, a_ref, w1_ref, b1_ref, w2_ref, b2_ref, g_ref, o_ref):
    i = pl.program_id(0)
    h = h_ref[...] + a_ref[...]
    h = jnp.dot(h, w1_ref[...], preferred_element_type=jnp.float32) + b1_ref[...]
    h = jnp.maximum(h, 0.0)
    h = jnp.dot(h, w2_ref[...], preferred_element_type=jnp.float32) + b2_ref[...]
    h = jnp.maximum(h, 0.0)
    gids = g_ref[0, 0, :]
    onehot = (gids[:, None] == jax.lax.broadcasted_iota(jnp.int32, (1, G), 1)
              ).astype(jnp.float32)
    part = jnp.dot(onehot.T, h, preferred_element_type=jnp.float32)

    @pl.when(i == 0)
    def _():
        o_ref[...] = jnp.zeros_like(o_ref)

    o_ref[...] += part


def _mlp_pool(h, agg, W1p, b1, W2p, b2, g3):
    return pl.pallas_call(
        _mlp_pool_body,
        grid=(NSTEP,),
        in_specs=[
            pl.BlockSpec((ROWS, H), lambda i: (i, _z())),
            pl.BlockSpec((ROWS, H), lambda i: (i, _z())),
            pl.BlockSpec((H, H), lambda i: (_z(), _z())),
            pl.BlockSpec((1, H), lambda i: (_z(), _z())),
            pl.BlockSpec((H, H), lambda i: (_z(), _z())),
            pl.BlockSpec((1, H), lambda i: (_z(), _z())),
            pl.BlockSpec((1, 1, ROWS), lambda i: (i, _z(), _z())),
        ],
        out_specs=pl.BlockSpec((G, H), lambda i: (_z(), _z())),
        out_shape=jax.ShapeDtypeStruct((G, H), jnp.float32),
    )(h, agg, W1p, b1.reshape(1, H), W2p, b2.reshape(1, H), g3)


def _head_body(h_ref, w1_ref, b1_ref, w2_ref, b2_ref, o_ref):
    h = jnp.dot(h_ref[...], w1_ref[...], preferred_element_type=jnp.float32) + b1_ref[...]
    h = jnp.maximum(h, 0.0)
    z = jnp.dot(h, w2_ref[...], preferred_element_type=jnp.float32) + b2_ref[...]
    m = jnp.max(z, axis=1, keepdims=True)
    lse = jnp.log(jnp.sum(jnp.exp(z - m), axis=1, keepdims=True)) + m
    o_ref[...] = z - lse


def _head(hg, lin1_W, lin1_b, lin2_W, lin2_b):
    return pl.pallas_call(
        _head_body,
        out_shape=jax.ShapeDtypeStruct((G, H), jnp.float32),
    )(hg, lin1_W, lin1_b.reshape(1, H), lin2_W, lin2_b.reshape(1, H))


def kernel(x, edge_index, node_to_subgraph, subgraph_to_graph,
           c1_W1, c1_b1, c1_W2, c1_b2,
           c2_W1, c2_b1, c2_W2, c2_b2,
           c3_W1, c3_b1, c3_W2, c3_b2,
           lin1_W, lin1_b, lin2_W, lin2_b):
    src = edge_index[0].astype(jnp.int32)
    dst = edge_index[1].astype(jnp.int32)
    npad = EPAD - E
    pad_src = (jnp.arange(npad, dtype=jnp.int32) * 97) % N
    pad_dst = N + (jnp.arange(npad, dtype=jnp.int32) % (NPAD - N))
    src_pad = jnp.concatenate([src, pad_src])
    dst_pad = jnp.concatenate([dst, pad_dst])
    ei_win = jnp.stack([src_pad.reshape(NWTOT, W2),
                        dst_pad.reshape(NWTOT, W2)], axis=1)

    n2s_pad = jnp.concatenate([node_to_subgraph.astype(jnp.int32),
                               jnp.zeros((NPAD - N,), jnp.int32)])
    s2g_pad = jnp.concatenate([subgraph_to_graph.astype(jnp.int32),
                               jnp.zeros((SPAD - S,), jnp.int32)])

    n2g_pad = _sc_n2g(n2s_pad, s2g_pad)
    g3 = n2g_pad[:N].reshape(NSTEP, 1, ROWS)

    y0 = _xw(x, c1_W1)
    agg1 = _sc_agg(y0, ei_win)
    h1 = _mlp_pre(y0, agg1, c1_b1, c1_W2, c1_b2)
    agg2 = _sc_agg(h1, ei_win)
    h2 = _mlp(h1, agg2, c2_W1, c2_b1, c2_W2, c2_b2)
    agg3 = _sc_agg(h2, ei_win)
    hg = _mlp_pool(h2, agg3, c3_W1, c3_b1, c3_W2, c3_b2, g3)
    return _head(hg, lin1_W, lin1_b, lin2_W, lin2_b)
